# Initial kernel scaffold; baseline (speedup 1.0000x reference)
import jax
import jax.numpy as jnp
from jax.experimental import pallas as pl


def kernel(x, edge_index, W_self_0, W_neigh_0, b_0, W_self_1, W_neigh_1, b_1):
    raise NotImplementedError("write your pallas kernel here")


# speedup vs baseline: 7.7964x; 7.7964x over previous
import functools

import jax
import jax.numpy as jnp
from jax import lax
from jax.experimental import pallas as pl
from jax.experimental.pallas import tpu as pltpu
from jax.experimental.pallas import tpu_sc as plsc

N_NODES = 10000
N_EDGES = 320000
D_IN = 128
D_HID = 128
N_CLASSES = 47

NP = 10240
W0 = 144
W1 = 48

NC = 2
NS = 16
NW = NC * NS
EPW = N_EDGES // NW
G = 125
NCH = EPW // G
RPT = NP // NS
ZR = 128
IGB = 16


def _make_edge_agg(width):
    mesh = plsc.VectorSubcoreMesh(core_axis_name="c", subcore_axis_name="s")

    @functools.partial(
        pl.kernel,
        mesh=mesh,
        compiler_params=pltpu.CompilerParams(use_tc_tiling_on_sc=False),
        out_type=jax.ShapeDtypeStruct((NC, NP, width), jnp.float32),
        scratch_types=[
            pltpu.VMEM((IGB, G), jnp.int32),
            pltpu.VMEM((IGB, G), jnp.int32),
            pltpu.VMEM((G, width), jnp.float32),
            pltpu.VMEM_SHARED((NP, width), jnp.float32),
            pltpu.SemaphoreType.DMA,
        ],
    )
    def edge_agg(tab_hbm, src_hbm, dst_hbm, zeros_hbm, out_hbm,
                 src_v, dst_v, rows_v, acc_sh, sem):
        c = lax.axis_index("c")
        s = lax.axis_index("s")
        wid = s * NC + c
        ebase = pl.multiple_of(wid * NCH, 8)
        rbase = pl.multiple_of(s * RPT, 8)

        for j in range(RPT // ZR):
            pltpu.sync_copy(zeros_hbm, acc_sh.at[pl.ds(rbase + j * ZR, ZR)])
        plsc.subcore_barrier()

        for g in range(NCH // IGB):
            pltpu.sync_copy(src_hbm.at[pl.ds(ebase + g * IGB, IGB)], src_v)
            pltpu.sync_copy(dst_hbm.at[pl.ds(ebase + g * IGB, IGB)], dst_v)

            def body(j, carry):
                pltpu.async_copy(tab_hbm.at[src_v.at[j]], rows_v, sem).wait()
                pltpu.sync_copy(rows_v, acc_sh.at[dst_v.at[j]], add=True)
                return carry

            lax.fori_loop(0, IGB, body, 0)
        plsc.subcore_barrier()

        for j in range(RPT // ZR):
            sl = pl.ds(rbase + j * ZR, ZR)
            pltpu.sync_copy(acc_sh.at[sl], out_hbm.at[c, sl])

    return edge_agg


_edge_agg_l0 = _make_edge_agg(W0)
_edge_agg_l1 = _make_edge_agg(W1)


def _tc_main_body(xa_ref, p0_ref, p1_ref, ws0_ref, wn0_ref, b0_ref,
                  wn1_ref, ws1_ref, b1_ref, proj_ref, self_ref, inv_ref):
    agg = p0_ref[...] + p1_ref[...]
    deg = agg[:, D_IN:D_IN + 1]
    inv = 1.0 / jnp.maximum(deg, 1.0)
    h_neigh = agg[:, :D_IN] * inv
    x = xa_ref[:, :D_IN]
    h1 = x @ ws0_ref[...] + h_neigh @ wn0_ref[...] + b0_ref[...]
    h1 = jnp.maximum(h1, 0.0)
    proj_ref[...] = h1 @ wn1_ref[...]
    self_ref[...] = h1 @ ws1_ref[...] + b1_ref[...]
    inv_ref[...] = inv


def _tc_epilogue_body(self_ref, a0_ref, a1_ref, inv_ref, out_ref):
    agg = a0_ref[...] + a1_ref[...]
    out_ref[...] = self_ref[...] + agg * inv_ref[...]


def kernel(x, edge_index, W_self_0, W_neigh_0, b_0, W_self_1, W_neigh_1, b_1):
    src = edge_index[0].astype(jnp.int32).reshape(N_EDGES // G, G)
    dst = edge_index[1].astype(jnp.int32).reshape(N_EDGES // G, G)

    x_aug = jnp.zeros((NP, W0), jnp.float32)
    x_aug = x_aug.at[:N_NODES, :D_IN].set(x).at[:N_NODES, D_IN].set(1.0)
    zeros0 = jnp.zeros((ZR, W0), jnp.float32)
    zeros1 = jnp.zeros((ZR, W1), jnp.float32)

    part0 = _edge_agg_l0(x_aug, src, dst, zeros0)

    wn1p = jnp.zeros((D_HID, W1), jnp.float32).at[:, :N_CLASSES].set(W_neigh_1)
    ws1p = jnp.zeros((D_HID, W1), jnp.float32).at[:, :N_CLASSES].set(W_self_1)
    b1p = jnp.zeros((1, W1), jnp.float32).at[0, :N_CLASSES].set(b_1)

    BR = 1024
    grid = (NP // BR,)
    proj, self1, inv = pl.pallas_call(
        _tc_main_body,
        grid=grid,
        in_specs=[
            pl.BlockSpec((BR, W0), lambda i: (i, 0)),
            pl.BlockSpec((BR, W0), lambda i: (i, 0)),
            pl.BlockSpec((BR, W0), lambda i: (i, 0)),
            pl.BlockSpec((D_IN, D_HID), lambda i: (0, 0)),
            pl.BlockSpec((D_IN, D_HID), lambda i: (0, 0)),
            pl.BlockSpec((1, D_HID), lambda i: (0, 0)),
            pl.BlockSpec((D_HID, W1), lambda i: (0, 0)),
            pl.BlockSpec((D_HID, W1), lambda i: (0, 0)),
            pl.BlockSpec((1, W1), lambda i: (0, 0)),
        ],
        out_specs=[
            pl.BlockSpec((BR, W1), lambda i: (i, 0)),
            pl.BlockSpec((BR, W1), lambda i: (i, 0)),
            pl.BlockSpec((BR, 1), lambda i: (i, 0)),
        ],
        out_shape=[
            jax.ShapeDtypeStruct((NP, W1), jnp.float32),
            jax.ShapeDtypeStruct((NP, W1), jnp.float32),
            jax.ShapeDtypeStruct((NP, 1), jnp.float32),
        ],
    )(x_aug, part0[0], part0[1], W_self_0, W_neigh_0, b_0.reshape(1, D_HID),
      wn1p, ws1p, b1p)

    part1 = _edge_agg_l1(proj, src, dst, zeros1)

    out48 = pl.pallas_call(
        _tc_epilogue_body,
        grid=grid,
        in_specs=[
            pl.BlockSpec((BR, W1), lambda i: (i, 0)),
            pl.BlockSpec((BR, W1), lambda i: (i, 0)),
            pl.BlockSpec((BR, W1), lambda i: (i, 0)),
            pl.BlockSpec((BR, 1), lambda i: (i, 0)),
        ],
        out_specs=pl.BlockSpec((BR, W1), lambda i: (i, 0)),
        out_shape=jax.ShapeDtypeStruct((NP, W1), jnp.float32),
    )(self1, part1[0], part1[1], inv)

    return out48[:N_NODES, :N_CLASSES]


# speedup vs baseline: 9.1660x; 1.1757x over previous
import functools

import jax
import jax.numpy as jnp
from jax import lax
from jax.experimental import pallas as pl
from jax.experimental.pallas import tpu as pltpu
from jax.experimental.pallas import tpu_sc as plsc

N_NODES = 10000
N_EDGES = 320000
D_IN = 128
D_HID = 128
N_CLASSES = 47

W0 = 144
W1 = 48

NC = 2
NS = 16
NW = NC * NS
EPW = N_EDGES // NW
G = 125
NCH = EPW // G
IGB = 16
RPT_A = 632
RPT_B = N_NODES - 15 * RPT_A


def _make_edge_agg(width):
    mesh = plsc.VectorSubcoreMesh(core_axis_name="c", subcore_axis_name="s")

    @functools.partial(
        pl.kernel,
        mesh=mesh,
        compiler_params=pltpu.CompilerParams(use_tc_tiling_on_sc=False),
        out_type=jax.ShapeDtypeStruct((NC, N_NODES, width), jnp.float32),
        scratch_types=[
            pltpu.VMEM((IGB, G), jnp.int32),
            pltpu.VMEM((IGB, G), jnp.int32),
            pltpu.VMEM((G, width), jnp.float32),
            pltpu.VMEM((G, width), jnp.float32),
            pltpu.VMEM_SHARED((N_NODES, width), jnp.float32),
            pltpu.SemaphoreType.DMA,
            pltpu.SemaphoreType.DMA,
        ],
    )
    def edge_agg(tab_hbm, src_hbm, dst_hbm, zeros_hbm, out_hbm,
                 src_v, dst_v, rows_a, rows_b, acc_sh, sem_a, sem_b):
        c = lax.axis_index("c")
        s = lax.axis_index("s")
        wid = s * NC + c
        ebase = pl.multiple_of(wid * NCH, 8)
        rbase = pl.multiple_of(s * RPT_A, 8)

        @pl.when(s < NS - 1)
        def _():
            pltpu.sync_copy(zeros_hbm, acc_sh.at[pl.ds(rbase, RPT_A)])

        @pl.when(s == NS - 1)
        def _():
            pltpu.sync_copy(zeros_hbm.at[pl.ds(0, RPT_B)],
                            acc_sh.at[pl.ds(rbase, RPT_B)])

        plsc.subcore_barrier()

        for g in range(NCH // IGB):
            pltpu.sync_copy(src_hbm.at[pl.ds(ebase + g * IGB, IGB)], src_v)
            pltpu.sync_copy(dst_hbm.at[pl.ds(ebase + g * IGB, IGB)], dst_v)

            def pair(i, carry):
                ca = pltpu.async_copy(tab_hbm.at[src_v.at[2 * i]],
                                      rows_a, sem_a)
                cb = pltpu.async_copy(tab_hbm.at[src_v.at[2 * i + 1]],
                                      rows_b, sem_b)
                ca.wait()
                pltpu.sync_copy(rows_a, acc_sh.at[dst_v.at[2 * i]], add=True)
                cb.wait()
                pltpu.sync_copy(rows_b, acc_sh.at[dst_v.at[2 * i + 1]],
                                add=True)
                return carry

            lax.fori_loop(0, IGB // 2, pair, 0)
        plsc.subcore_barrier()

        @pl.when(s < NS - 1)
        def _():
            pltpu.sync_copy(acc_sh.at[pl.ds(rbase, RPT_A)],
                            out_hbm.at[c, pl.ds(rbase, RPT_A)])

        @pl.when(s == NS - 1)
        def _():
            pltpu.sync_copy(acc_sh.at[pl.ds(rbase, RPT_B)],
                            out_hbm.at[c, pl.ds(rbase, RPT_B)])

    return edge_agg


_edge_agg_l0 = _make_edge_agg(W0)
_edge_agg_l1 = _make_edge_agg(W1)


def _tc_main_body(xa_ref, p0_ref, p1_ref, ws0_ref, wn0_ref, b0_ref,
                  wn1_ref, ws1_ref, b1_ref, proj_ref, self_ref, inv_ref):
    agg = p0_ref[...] + p1_ref[...]
    deg = agg[:, D_IN:D_IN + 1]
    inv = 1.0 / jnp.maximum(deg, 1.0)
    h_neigh = agg[:, :D_IN] * inv
    x = xa_ref[:, :D_IN]
    h1 = x @ ws0_ref[...] + h_neigh @ wn0_ref[...] + b0_ref[...]
    h1 = jnp.maximum(h1, 0.0)
    proj_ref[...] = h1 @ wn1_ref[...]
    self_ref[...] = h1 @ ws1_ref[...] + b1_ref[...]
    inv_ref[...] = inv


def _tc_epilogue_body(self_ref, a0_ref, a1_ref, inv_ref, out_ref):
    agg = a0_ref[...] + a1_ref[...]
    out_ref[...] = self_ref[...] + agg * inv_ref[...]


def kernel(x, edge_index, W_self_0, W_neigh_0, b_0, W_self_1, W_neigh_1, b_1):
    src = edge_index[0].astype(jnp.int32).reshape(N_EDGES // G, G)
    dst = edge_index[1].astype(jnp.int32).reshape(N_EDGES // G, G)

    x_aug = jnp.zeros((N_NODES, W0), jnp.float32)
    x_aug = x_aug.at[:, :D_IN].set(x).at[:, D_IN].set(1.0)
    zeros0 = jnp.zeros((RPT_A, W0), jnp.float32)
    zeros1 = jnp.zeros((RPT_A, W1), jnp.float32)

    part0 = _edge_agg_l0(x_aug, src, dst, zeros0)

    wn1p = jnp.zeros((D_HID, W1), jnp.float32).at[:, :N_CLASSES].set(W_neigh_1)
    ws1p = jnp.zeros((D_HID, W1), jnp.float32).at[:, :N_CLASSES].set(W_self_1)
    b1p = jnp.zeros((1, W1), jnp.float32).at[0, :N_CLASSES].set(b_1)

    BR = 1000
    grid = (N_NODES // BR,)
    proj, self1, inv = pl.pallas_call(
        _tc_main_body,
        grid=grid,
        in_specs=[
            pl.BlockSpec((BR, W0), lambda i: (i, 0)),
            pl.BlockSpec((BR, W0), lambda i: (i, 0)),
            pl.BlockSpec((BR, W0), lambda i: (i, 0)),
            pl.BlockSpec((D_IN, D_HID), lambda i: (0, 0)),
            pl.BlockSpec((D_IN, D_HID), lambda i: (0, 0)),
            pl.BlockSpec((1, D_HID), lambda i: (0, 0)),
            pl.BlockSpec((D_HID, W1), lambda i: (0, 0)),
            pl.BlockSpec((D_HID, W1), lambda i: (0, 0)),
            pl.BlockSpec((1, W1), lambda i: (0, 0)),
        ],
        out_specs=[
            pl.BlockSpec((BR, W1), lambda i: (i, 0)),
            pl.BlockSpec((BR, W1), lambda i: (i, 0)),
            pl.BlockSpec((BR, 1), lambda i: (i, 0)),
        ],
        out_shape=[
            jax.ShapeDtypeStruct((N_NODES, W1), jnp.float32),
            jax.ShapeDtypeStruct((N_NODES, W1), jnp.float32),
            jax.ShapeDtypeStruct((N_NODES, 1), jnp.float32),
        ],
    )(x_aug, part0[0], part0[1], W_self_0, W_neigh_0, b_0.reshape(1, D_HID),
      wn1p, ws1p, b1p)

    part1 = _edge_agg_l1(proj, src, dst, zeros1)

    out48 = pl.pallas_call(
        _tc_epilogue_body,
        grid=grid,
        in_specs=[
            pl.BlockSpec((BR, W1), lambda i: (i, 0)),
            pl.BlockSpec((BR, W1), lambda i: (i, 0)),
            pl.BlockSpec((BR, W1), lambda i: (i, 0)),
            pl.BlockSpec((BR, 1), lambda i: (i, 0)),
        ],
        out_specs=pl.BlockSpec((BR, W1), lambda i: (i, 0)),
        out_shape=jax.ShapeDtypeStruct((N_NODES, W1), jnp.float32),
    )(self1, part1[0], part1[1], inv)

    return out48[:, :N_CLASSES]


# speedup vs baseline: 11.1132x; 1.2124x over previous
import functools

import jax
import jax.numpy as jnp
from jax import lax
from jax.experimental import pallas as pl
from jax.experimental.pallas import tpu as pltpu
from jax.experimental.pallas import tpu_sc as plsc

N_NODES = 10000
N_EDGES = 320000
D_IN = 128
D_HID = 128
N_CLASSES = 47

W0 = 144
W1 = 48

NC = 2
NS = 16
NW = NC * NS
EPW = N_EDGES // NW
G = 125
NCH = EPW // G
IGB = 16
RPT_A = 632
RPT_B = N_NODES - 15 * RPT_A


def _make_edge_agg(width):
    mesh = plsc.VectorSubcoreMesh(core_axis_name="c", subcore_axis_name="s")

    @functools.partial(
        pl.kernel,
        mesh=mesh,
        compiler_params=pltpu.CompilerParams(use_tc_tiling_on_sc=False),
        out_type=jax.ShapeDtypeStruct((NC, N_NODES, width), jnp.float32),
        scratch_types=[
            pltpu.VMEM((IGB, G), jnp.int32),
            pltpu.VMEM((IGB, G), jnp.int32),
            pltpu.VMEM((G, width), jnp.float32),
            pltpu.VMEM((G, width), jnp.float32),
            pltpu.VMEM_SHARED((N_NODES, width), jnp.float32),
            pltpu.SemaphoreType.DMA,
            pltpu.SemaphoreType.DMA,
        ],
    )
    def edge_agg(tab_hbm, src_hbm, dst_hbm, zeros_hbm, out_hbm,
                 src_v, dst_v, rows_a, rows_b, acc_sh, sem_a, sem_b):
        c = lax.axis_index("c")
        s = lax.axis_index("s")
        wid = s * NC + c
        ebase = pl.multiple_of(wid * NCH, 8)
        rbase = pl.multiple_of(s * RPT_A, 8)

        @pl.when(s < NS - 1)
        def _():
            pltpu.sync_copy(zeros_hbm, acc_sh.at[pl.ds(rbase, RPT_A)])

        @pl.when(s == NS - 1)
        def _():
            pltpu.sync_copy(zeros_hbm.at[pl.ds(0, RPT_B)],
                            acc_sh.at[pl.ds(rbase, RPT_B)])

        plsc.subcore_barrier()

        for g in range(NCH // IGB):
            pltpu.sync_copy(src_hbm.at[pl.ds(ebase + g * IGB, IGB)], src_v)
            pltpu.sync_copy(dst_hbm.at[pl.ds(ebase + g * IGB, IGB)], dst_v)

            pltpu.async_copy(tab_hbm.at[src_v.at[0]], rows_a, sem_a)

            def pair(i, carry):
                pltpu.async_copy(tab_hbm.at[src_v.at[2 * i + 1]],
                                 rows_b, sem_b)
                pltpu.make_async_copy(tab_hbm.at[src_v.at[0]],
                                      rows_a, sem_a).wait()
                pltpu.sync_copy(rows_a, acc_sh.at[dst_v.at[2 * i]], add=True)

                @pl.when(i < IGB // 2 - 1)
                def _():
                    pltpu.async_copy(tab_hbm.at[src_v.at[2 * i + 2]],
                                     rows_a, sem_a)

                pltpu.make_async_copy(tab_hbm.at[src_v.at[0]],
                                      rows_b, sem_b).wait()
                pltpu.sync_copy(rows_b, acc_sh.at[dst_v.at[2 * i + 1]],
                                add=True)
                return carry

            lax.fori_loop(0, IGB // 2, pair, 0)
        plsc.subcore_barrier()

        @pl.when(s < NS - 1)
        def _():
            pltpu.sync_copy(acc_sh.at[pl.ds(rbase, RPT_A)],
                            out_hbm.at[c, pl.ds(rbase, RPT_A)])

        @pl.when(s == NS - 1)
        def _():
            pltpu.sync_copy(acc_sh.at[pl.ds(rbase, RPT_B)],
                            out_hbm.at[c, pl.ds(rbase, RPT_B)])

    return edge_agg


_edge_agg_l0 = _make_edge_agg(W0)
_edge_agg_l1 = _make_edge_agg(W1)


def _tc_main_body(xa_ref, p0_ref, p1_ref, ws0_ref, wn0_ref, b0_ref,
                  wn1_ref, ws1_ref, b1_ref, proj_ref, self_ref, inv_ref):
    agg = p0_ref[0] + p1_ref[0]
    deg = agg[:, D_IN:D_IN + 1]
    inv = 1.0 / jnp.maximum(deg, 1.0)
    h_neigh = agg[:, :D_IN] * inv
    x = xa_ref[:, :D_IN]
    h1 = x @ ws0_ref[...] + h_neigh @ wn0_ref[...] + b0_ref[...]
    h1 = jnp.maximum(h1, 0.0)
    proj_ref[...] = h1 @ wn1_ref[...]
    self_ref[...] = h1 @ ws1_ref[...] + b1_ref[...]
    inv_ref[...] = inv


def _tc_epilogue_body(self_ref, a0_ref, a1_ref, inv_ref, out_ref):
    agg = a0_ref[0] + a1_ref[0]
    out_ref[...] = self_ref[...] + agg * inv_ref[...]


def kernel(x, edge_index, W_self_0, W_neigh_0, b_0, W_self_1, W_neigh_1, b_1):
    src = edge_index[0].astype(jnp.int32).reshape(N_EDGES // G, G)
    dst = edge_index[1].astype(jnp.int32).reshape(N_EDGES // G, G)

    x_aug = jnp.zeros((N_NODES, W0), jnp.float32)
    x_aug = x_aug.at[:, :D_IN].set(x).at[:, D_IN].set(1.0)
    zeros0 = jnp.zeros((RPT_A, W0), jnp.float32)
    zeros1 = jnp.zeros((RPT_A, W1), jnp.float32)

    part0 = _edge_agg_l0(x_aug, src, dst, zeros0)

    wn1p = jnp.zeros((D_HID, W1), jnp.float32).at[:, :N_CLASSES].set(W_neigh_1)
    ws1p = jnp.zeros((D_HID, W1), jnp.float32).at[:, :N_CLASSES].set(W_self_1)
    b1p = jnp.zeros((1, W1), jnp.float32).at[0, :N_CLASSES].set(b_1)

    BR = 1000
    grid = (N_NODES // BR,)
    proj, self1, inv = pl.pallas_call(
        _tc_main_body,
        grid=grid,
        in_specs=[
            pl.BlockSpec((BR, W0), lambda i: (i, 0)),
            pl.BlockSpec((1, BR, W0), lambda i: (0, i, 0)),
            pl.BlockSpec((1, BR, W0), lambda i: (1, i, 0)),
            pl.BlockSpec((D_IN, D_HID), lambda i: (0, 0)),
            pl.BlockSpec((D_IN, D_HID), lambda i: (0, 0)),
            pl.BlockSpec((1, D_HID), lambda i: (0, 0)),
            pl.BlockSpec((D_HID, W1), lambda i: (0, 0)),
            pl.BlockSpec((D_HID, W1), lambda i: (0, 0)),
            pl.BlockSpec((1, W1), lambda i: (0, 0)),
        ],
        out_specs=[
            pl.BlockSpec((BR, W1), lambda i: (i, 0)),
            pl.BlockSpec((BR, W1), lambda i: (i, 0)),
            pl.BlockSpec((BR, 1), lambda i: (i, 0)),
        ],
        out_shape=[
            jax.ShapeDtypeStruct((N_NODES, W1), jnp.float32),
            jax.ShapeDtypeStruct((N_NODES, W1), jnp.float32),
            jax.ShapeDtypeStruct((N_NODES, 1), jnp.float32),
        ],
    )(x_aug, part0, part0, W_self_0, W_neigh_0, b_0.reshape(1, D_HID),
      wn1p, ws1p, b1p)

    part1 = _edge_agg_l1(proj, src, dst, zeros1)

    out48 = pl.pallas_call(
        _tc_epilogue_body,
        grid=grid,
        in_specs=[
            pl.BlockSpec((BR, W1), lambda i: (i, 0)),
            pl.BlockSpec((1, BR, W1), lambda i: (0, i, 0)),
            pl.BlockSpec((1, BR, W1), lambda i: (1, i, 0)),
            pl.BlockSpec((BR, 1), lambda i: (i, 0)),
        ],
        out_specs=pl.BlockSpec((BR, W1), lambda i: (i, 0)),
        out_shape=jax.ShapeDtypeStruct((N_NODES, W1), jnp.float32),
    )(self1, part1, part1, inv)

    return out48[:, :N_CLASSES]


# speedup vs baseline: 11.1171x; 1.0004x over previous
import functools

import jax
import jax.numpy as jnp
from jax import lax
from jax.experimental import pallas as pl
from jax.experimental.pallas import tpu as pltpu
from jax.experimental.pallas import tpu_sc as plsc

N_NODES = 10000
N_EDGES = 320000
D_IN = 128
D_HID = 128
N_CLASSES = 47

W0 = 144
W1 = 48

NC = 2
NS = 16
NW = NC * NS
EPW = N_EDGES // NW
G = 125
NCH = EPW // G
IGB = 16
RPT_A = 632
RPT_B = N_NODES - 15 * RPT_A


def _make_edge_agg(width):
    mesh = plsc.VectorSubcoreMesh(core_axis_name="c", subcore_axis_name="s")

    @functools.partial(
        pl.kernel,
        mesh=mesh,
        compiler_params=pltpu.CompilerParams(
            use_tc_tiling_on_sc=False,
            disable_bounds_checks=True,
            disable_semaphore_checks=True,
        ),
        out_type=jax.ShapeDtypeStruct((NC, N_NODES, width), jnp.float32),
        scratch_types=[
            pltpu.VMEM((IGB, G), jnp.int32),
            pltpu.VMEM((IGB, G), jnp.int32),
            pltpu.VMEM((G, width), jnp.float32),
            pltpu.VMEM((G, width), jnp.float32),
            pltpu.VMEM_SHARED((N_NODES, width), jnp.float32),
            pltpu.SemaphoreType.DMA,
            pltpu.SemaphoreType.DMA,
        ],
    )
    def edge_agg(tab_hbm, src_hbm, dst_hbm, zeros_hbm, out_hbm,
                 src_v, dst_v, rows_a, rows_b, acc_sh, sem_a, sem_b):
        c = lax.axis_index("c")
        s = lax.axis_index("s")
        wid = s * NC + c
        ebase = pl.multiple_of(wid * NCH, 8)
        rbase = pl.multiple_of(s * RPT_A, 8)

        @pl.when(s < NS - 1)
        def _():
            pltpu.sync_copy(zeros_hbm, acc_sh.at[pl.ds(rbase, RPT_A)])

        @pl.when(s == NS - 1)
        def _():
            pltpu.sync_copy(zeros_hbm.at[pl.ds(0, RPT_B)],
                            acc_sh.at[pl.ds(rbase, RPT_B)])

        plsc.subcore_barrier()

        for g in range(NCH // IGB):
            pltpu.sync_copy(src_hbm.at[pl.ds(ebase + g * IGB, IGB)], src_v)
            pltpu.sync_copy(dst_hbm.at[pl.ds(ebase + g * IGB, IGB)], dst_v)

            pltpu.async_copy(tab_hbm.at[src_v.at[0]], rows_a, sem_a)

            def pair(i, carry):
                pltpu.async_copy(tab_hbm.at[src_v.at[2 * i + 1]],
                                 rows_b, sem_b)
                pltpu.make_async_copy(tab_hbm.at[src_v.at[0]],
                                      rows_a, sem_a).wait()
                pltpu.sync_copy(rows_a, acc_sh.at[dst_v.at[2 * i]], add=True)

                @pl.when(i < IGB // 2 - 1)
                def _():
                    pltpu.async_copy(tab_hbm.at[src_v.at[2 * i + 2]],
                                     rows_a, sem_a)

                pltpu.make_async_copy(tab_hbm.at[src_v.at[0]],
                                      rows_b, sem_b).wait()
                pltpu.sync_copy(rows_b, acc_sh.at[dst_v.at[2 * i + 1]],
                                add=True)
                return carry

            lax.fori_loop(0, IGB // 2, pair, 0)
        plsc.subcore_barrier()

        @pl.when(s < NS - 1)
        def _():
            pltpu.sync_copy(acc_sh.at[pl.ds(rbase, RPT_A)],
                            out_hbm.at[c, pl.ds(rbase, RPT_A)])

        @pl.when(s == NS - 1)
        def _():
            pltpu.sync_copy(acc_sh.at[pl.ds(rbase, RPT_B)],
                            out_hbm.at[c, pl.ds(rbase, RPT_B)])

    return edge_agg


_edge_agg_l0 = _make_edge_agg(W0)
_edge_agg_l1 = _make_edge_agg(W1)


def _tc_main_body(xa_ref, p0_ref, p1_ref, ws0_ref, wn0_ref, b0_ref,
                  wn1_ref, ws1_ref, b1_ref, proj_ref, self_ref, inv_ref):
    agg = p0_ref[0] + p1_ref[0]
    deg = agg[:, D_IN:D_IN + 1]
    inv = 1.0 / jnp.maximum(deg, 1.0)
    h_neigh = agg[:, :D_IN] * inv
    x = xa_ref[:, :D_IN]
    h1 = x @ ws0_ref[...] + h_neigh @ wn0_ref[...] + b0_ref[...]
    h1 = jnp.maximum(h1, 0.0)
    proj_ref[...] = h1 @ wn1_ref[...]
    self_ref[...] = h1 @ ws1_ref[...] + b1_ref[...]
    inv_ref[...] = inv


def _tc_epilogue_body(self_ref, a0_ref, a1_ref, inv_ref, out_ref):
    agg = a0_ref[0] + a1_ref[0]
    out_ref[...] = self_ref[...] + agg * inv_ref[...]


def kernel(x, edge_index, W_self_0, W_neigh_0, b_0, W_self_1, W_neigh_1, b_1):
    src = edge_index[0].astype(jnp.int32).reshape(N_EDGES // G, G)
    dst = edge_index[1].astype(jnp.int32).reshape(N_EDGES // G, G)

    x_aug = jnp.zeros((N_NODES, W0), jnp.float32)
    x_aug = x_aug.at[:, :D_IN].set(x).at[:, D_IN].set(1.0)
    zeros0 = jnp.zeros((RPT_A, W0), jnp.float32)
    zeros1 = jnp.zeros((RPT_A, W1), jnp.float32)

    part0 = _edge_agg_l0(x_aug, src, dst, zeros0)

    wn1p = jnp.zeros((D_HID, W1), jnp.float32).at[:, :N_CLASSES].set(W_neigh_1)
    ws1p = jnp.zeros((D_HID, W1), jnp.float32).at[:, :N_CLASSES].set(W_self_1)
    b1p = jnp.zeros((1, W1), jnp.float32).at[0, :N_CLASSES].set(b_1)

    BR = 1000
    grid = (N_NODES // BR,)
    proj, self1, inv = pl.pallas_call(
        _tc_main_body,
        grid=grid,
        in_specs=[
            pl.BlockSpec((BR, W0), lambda i: (i, 0)),
            pl.BlockSpec((1, BR, W0), lambda i: (0, i, 0)),
            pl.BlockSpec((1, BR, W0), lambda i: (1, i, 0)),
            pl.BlockSpec((D_IN, D_HID), lambda i: (0, 0)),
            pl.BlockSpec((D_IN, D_HID), lambda i: (0, 0)),
            pl.BlockSpec((1, D_HID), lambda i: (0, 0)),
            pl.BlockSpec((D_HID, W1), lambda i: (0, 0)),
            pl.BlockSpec((D_HID, W1), lambda i: (0, 0)),
            pl.BlockSpec((1, W1), lambda i: (0, 0)),
        ],
        out_specs=[
            pl.BlockSpec((BR, W1), lambda i: (i, 0)),
            pl.BlockSpec((BR, W1), lambda i: (i, 0)),
            pl.BlockSpec((BR, 1), lambda i: (i, 0)),
        ],
        out_shape=[
            jax.ShapeDtypeStruct((N_NODES, W1), jnp.float32),
            jax.ShapeDtypeStruct((N_NODES, W1), jnp.float32),
            jax.ShapeDtypeStruct((N_NODES, 1), jnp.float32),
        ],
    )(x_aug, part0, part0, W_self_0, W_neigh_0, b_0.reshape(1, D_HID),
      wn1p, ws1p, b1p)

    part1 = _edge_agg_l1(proj, src, dst, zeros1)

    out48 = pl.pallas_call(
        _tc_epilogue_body,
        grid=grid,
        in_specs=[
            pl.BlockSpec((BR, W1), lambda i: (i, 0)),
            pl.BlockSpec((1, BR, W1), lambda i: (0, i, 0)),
            pl.BlockSpec((1, BR, W1), lambda i: (1, i, 0)),
            pl.BlockSpec((BR, 1), lambda i: (i, 0)),
        ],
        out_specs=pl.BlockSpec((BR, W1), lambda i: (i, 0)),
        out_shape=jax.ShapeDtypeStruct((N_NODES, W1), jnp.float32),
    )(self1, part1, part1, inv)

    return out48[:, :N_CLASSES]
